# baseline (speedup 1.0000x reference)
import functools

import jax
import jax.numpy as jnp
from jax import lax
from jax.experimental import pallas as pl
from jax.experimental.pallas import tpu as pltpu
from jax.experimental.pallas import tpu_sc as plsc

B, N, H = 32, 128, 32
NP = N + 1
JP = 144
NG = JP // 16
VS = 512
VE = 1537
RB = 8
NBLK = 17
TS = H // 2 + 1

_mesh = plsc.VectorSubcoreMesh(core_axis_name="c", subcore_axis_name="s")


@functools.partial(
    pl.kernel,
    mesh=_mesh,
    compiler_params=pltpu.CompilerParams(use_tc_tiling_on_sc=False,
                                         needs_layout_passes=False),
    out_type=jax.ShapeDtypeStruct((B, H, NP, NP), jnp.float32),
    scratch_types=[
        pltpu.VMEM(((VS + 1) * TS,), jnp.int32),
        pltpu.VMEM(((VE + 1) * TS,), jnp.int32),
        pltpu.VMEM((RB, N), jnp.int32),
        pltpu.VMEM((RB, N, 3), jnp.int32),
        pltpu.VMEM((RB, JP), jnp.float32),
        pltpu.VMEM((2, H, RB, NP), jnp.float32),
        pltpu.SemaphoreType.DMA,
        pltpu.SemaphoreType.DMA,
    ],
)
def _graph_attn_bias_sc(sp_h, ed_h, spi_h, ei_h, bias_h, out_h,
                        sp_tab, ed_tab, spi, ei, brow, obuf2, sem, osem):
    b = lax.axis_index("s") * 2 + lax.axis_index("c")
    pltpu.async_copy(sp_h, sp_tab, sem).wait()
    pltpu.async_copy(ed_h, ed_tab, sem).wait()
    c128 = jnp.full((16,), N, jnp.int32)
    hv = jax.lax.iota(jnp.int32, 16)

    def blk_body(blk, carry):
        r0 = jnp.minimum(blk * RB, NP - RB)
        p = jnp.bitwise_and(blk, 1)
        obuf = obuf2.at[p]
        rm = jnp.maximum(r0 - 1, 0)
        c1 = pltpu.async_copy(spi_h.at[b, pl.ds(rm, RB)], spi, sem)
        c2 = pltpu.async_copy(ei_h.at[b, pl.ds(rm, RB)], ei, sem)
        c3 = pltpu.async_copy(bias_h.at[b, pl.ds(r0, RB)], brow, sem)
        c1.wait()
        c2.wait()
        c3.wait()

        @pl.when(blk >= 2)
        def _drain_older():
            pltpu.make_async_copy(out_h.at[b, :, pl.ds(0, RB), :],
                                  obuf2.at[p], osem).wait()

        def gload2(tab, a):
            g = plsc.load_gather(tab, [a])
            lo, hi = plsc.unpack(plsc.bitcast(g, jnp.bfloat16),
                                 format=plsc.PackFormat.INTERLEAVED)
            return lo.astype(jnp.float32), hi.astype(jnp.float32)

        def row_body(rr, c2):
            rzv = jnp.broadcast_to(r0 + rr == 0, (16,))
            brr = jnp.zeros((16,), jnp.int32) + jnp.maximum(r0 + rr - 1 - rm,
                                                            0)
            for jg in range(N // 16):
                sl = pl.ds(jg * 16, 16)
                jvec = hv + jg * 16
                jm1 = jnp.maximum(jvec - 1, 0)
                sel = (jvec == 0) | rzv
                sidx = jnp.where(sel, VS, plsc.load_gather(spi, [brr, jm1]))
                bsp = jnp.clip(sidx, 0, VS) * TS
                eidx = [jnp.where(sel, VE,
                                  plsc.load_gather(
                                      ei, [brr, jm1,
                                           jnp.full((16,), k, jnp.int32)]))
                        for k in range(3)]
                be0, be1, be2 = (jnp.clip(x, 0, VE) * TS for x in eidx)
                bv = brow[rr, sl]

                @plsc.parallel_loop(0, H // 2, 1, unroll=4)
                def _h_loop(h, sl=sl, bsp=bsp, be0=be0, be1=be1, be2=be2,
                            bv=bv, rr=rr):
                    vl, vh = gload2(sp_tab, bsp + h)
                    e0l, e0h = gload2(ed_tab, be0 + h)
                    e1l, e1h = gload2(ed_tab, be1 + h)
                    e2l, e2h = gload2(ed_tab, be2 + h)
                    obuf[h, rr, sl] = bv + vl + (e0l + e1l + e2l) * (1.0 / 3.0)
                    obuf[h + H // 2, rr, sl] = (bv + vh
                                                + (e0h + e1h + e2h)
                                                * (1.0 / 3.0))
            rrv = jnp.zeros((16,), jnp.int32) + rr
            j127 = jnp.full((16,), N - 1, jnp.int32)
            isp = jnp.where(rzv, VS, plsc.load_gather(spi, [brr, j127]))
            isp = jnp.clip(isp, 0, VS) * TS
            ies = [jnp.where(rzv, VE,
                             plsc.load_gather(
                                 ei, [brr, j127,
                                      jnp.full((16,), k, jnp.int32)]))
                   for k in range(3)]
            ie0, ie1, ie2 = (jnp.clip(x, 0, VE) * TS for x in ies)
            bvc = plsc.load_gather(brow, [rrv, c128])
            vl, vh = gload2(sp_tab, isp + hv)
            e0l, e0h = gload2(ed_tab, ie0 + hv)
            e1l, e1h = gload2(ed_tab, ie1 + hv)
            e2l, e2h = gload2(ed_tab, ie2 + hv)
            plsc.store_scatter(obuf, [hv, rrv, c128],
                               bvc + vl + (e0l + e1l + e2l) * (1.0 / 3.0))
            plsc.store_scatter(obuf, [hv + H // 2, rrv, c128],
                               bvc + vh + (e0h + e1h + e2h) * (1.0 / 3.0))
            return c2

        lax.fori_loop(0, RB, row_body, 0)
        pltpu.async_copy(obuf, out_h.at[b, :, pl.ds(r0, RB), :], osem)
        return carry

    lax.fori_loop(0, NBLK, blk_body, 0)
    for _ in range(2):
        pltpu.make_async_copy(out_h.at[b, :, pl.ds(0, RB), :],
                              obuf2.at[0], osem).wait()


def kernel(attn_bias, spatial_pos, attn_edge_type, spatial_pos_table,
           edge_table, virtual_dist):
    f32 = jnp.float32
    sp_aug = jnp.concatenate(
        [spatial_pos_table.astype(f32), virtual_dist.astype(f32).reshape(1, H)],
        axis=0)
    ed_aug = jnp.concatenate(
        [edge_table.astype(f32), jnp.zeros((1, H), f32)], axis=0)

    def _pack(t):
        bb = jax.lax.bitcast_convert_type(t.astype(jnp.bfloat16), jnp.uint16)
        w = (bb[:, :H // 2].astype(jnp.uint32)
             | (bb[:, H // 2:].astype(jnp.uint32) << 16))
        w = jnp.pad(w, ((0, 0), (0, TS - H // 2)))
        return jax.lax.bitcast_convert_type(w, jnp.int32).reshape(-1)

    sp_aug = _pack(sp_aug)
    ed_aug = _pack(ed_aug)
    spi = spatial_pos.astype(jnp.int32)
    ei = attn_edge_type.astype(jnp.int32)
    biasp = jnp.pad(attn_bias.astype(f32), ((0, 0), (0, 0), (0, JP - NP)))
    return _graph_attn_bias_sc(sp_aug, ed_aug, spi, ei, biasp)
